# baseline (speedup 1.0000x reference)
import functools

import jax
import jax.numpy as jnp
from jax import lax
from jax.experimental import pallas as pl
from jax.experimental.pallas import tpu as pltpu
from jax.experimental.pallas import tpu_sc as plsc

N_USERS = 10000
N_ENT = 40000
NN = N_USERS + N_ENT
E = 800000
D = 64

NC = 2
NS = 16
HALF = NN // 2
ROWS_PER_TILE = 1564
ACC_ROWS = NS * ROWS_PER_TILE
TRASH = ACC_ROWS - 1

K = 64
CHUNKS_PER_TILE = 784
PER_TILE = CHUNKS_PER_TILE * K
E_PAD = NS * PER_TILE
C2 = CHUNKS_PER_TILE // 2

_mesh = plsc.VectorSubcoreMesh(core_axis_name="c", subcore_axis_name="s")


def _zero16():
    return jnp.zeros((16,), jnp.float32)


BLK = 8
NBLK = CHUNKS_PER_TILE // BLK
WBR = 68


@functools.partial(
    pl.kernel,
    out_type=jax.ShapeDtypeStruct((NN, D), jnp.float32),
    mesh=_mesh,
    scratch_types=[
        pltpu.VMEM((3, BLK, K), jnp.int32),
        pltpu.VMEM((3, BLK, K), jnp.int32),
        pltpu.VMEM((3, BLK, K), jnp.float32),
        pltpu.VMEM((K, 4 * D), jnp.uint8),
        pltpu.VMEM((K, 4 * D), jnp.uint8),
        pltpu.VMEM((K, D), jnp.float32),
        pltpu.VMEM((K, D), jnp.float32),
        pltpu.VMEM((WBR, D), jnp.float32),
        pltpu.VMEM_SHARED((ACC_ROWS, D), jnp.float32),
        pltpu.SemaphoreType.DMA,
        pltpu.SemaphoreType.DMA,
        pltpu.SemaphoreType.DMA,
        pltpu.SemaphoreType.DMA,
        pltpu.SemaphoreType.DMA,
        pltpu.SemaphoreType.DMA,
        pltpu.SemaphoreType.DMA,
    ],
    compiler_params=pltpu.CompilerParams(use_tc_tiling_on_sc=False, needs_layout_passes=False),
)
def _spmm(src_hbm, dst_hbm, val_hbm, feats_hbm, out_hbm,
          srcm, dstm, valm, rows0, rows1, rowsf0, rowsf1, wbuf, acc,
          msem0, msem1, msem2, gsem0, gsem1, ssem0, ssem1):
    c = lax.axis_index("c")
    s = lax.axis_index("s")
    lo = c * HALF
    row_base = s * CHUNKS_PER_TILE
    msems = (msem0, msem1, msem2)

    def _fire_meta(b, slot, sem):
        rb = row_base + b * BLK
        pltpu.async_copy(src_hbm.at[pl.ds(rb, BLK)], srcm.at[slot], sem)
        pltpu.async_copy(dst_hbm.at[pl.ds(rb, BLK)], dstm.at[slot], sem)
        pltpu.async_copy(val_hbm.at[pl.ds(rb, BLK)], valm.at[slot], sem)

    def _wait_meta(b, slot, sem):
        rb = row_base + b * BLK
        pltpu.make_async_copy(src_hbm.at[pl.ds(rb, BLK)], srcm.at[slot], sem).wait()
        pltpu.make_async_copy(dst_hbm.at[pl.ds(rb, BLK)], dstm.at[slot], sem).wait()
        pltpu.make_async_copy(val_hbm.at[pl.ds(rb, BLK)], valm.at[slot], sem).wait()

    def _wait_meta_dyn(b, slot):
        for i in range(3):
            @pl.when(slot == i)
            def _():
                _wait_meta(b, slot, msems[i])

    def _fire_meta_dyn(b, slot):
        for i in range(3):
            @pl.when(slot == i)
            def _():
                _fire_meta(b, slot, msems[i])

    def _mask(slot):
        for ci in range(BLK):
            for v in range(K // 16):
                sl = pl.ds(v * 16, 16)
                dl = dstm[slot, ci, sl] - lo
                inm = (dl >= 0) & (dl < HALF)
                dstm[slot, ci, sl] = jnp.where(inm, dl, TRASH)
                valm[slot, ci, sl] = jnp.where(inm, valm[slot, ci, sl], 0.0)

    def _fire_gather(t, rows, gsem):
        slot = lax.rem(lax.div(t, BLK), 3)
        ci = lax.rem(t, BLK)
        pltpu.async_copy(feats_hbm.at[srcm.at[slot, ci]], rows, gsem)

    def _drain_gather(rows, gsem):
        pltpu.make_async_copy(feats_hbm.at[srcm.at[0, 0]], rows, gsem).wait()

    def _scale(slot, ci, rows, rowsf):
        def _e16(i, _):
            vv = valm[slot, ci, pl.ds(i * 16, 16)]
            for de in range(16):
                e = i * 16 + de
                v = lax.gather(
                    vv, jnp.full((16, 1), de, jnp.int32),
                    lax.GatherDimensionNumbers(
                        offset_dims=(), collapsed_slice_dims=(0,),
                        start_index_map=(0,)),
                    slice_sizes=(1,),
                    mode=lax.GatherScatterMode.PROMISE_IN_BOUNDS)
                for j in range(D // 16):
                    r = plsc.bitcast(rows[e, pl.ds(j * 64, 64)], jnp.float32)
                    rowsf[e, pl.ds(j * 16, 16)] = r * v
            return 0
        lax.fori_loop(0, K // 16, _e16, 0)

    def _fire_scatter(slot, ci, rowsf, ssem):
        pltpu.async_copy(rowsf, acc.at[dstm.at[slot, ci]], ssem, add=True)

    def _wait_scatter(rowsf, ssem):
        pltpu.make_async_copy(rowsf, acc.at[dstm.at[0, 0]], ssem).wait()

    _fire_meta(0, 0, msem0)

    def _fill_z(r, _):
        for j in range(D // 16):
            wbuf[r, pl.ds(j * 16, 16)] = _zero16()
        return 0
    lax.fori_loop(0, WBR, _fill_z, 0)
    for r in range(ROWS_PER_TILE // WBR):
        pltpu.sync_copy(wbuf, acc.at[pl.ds(s * ROWS_PER_TILE + r * WBR, WBR)])
    plsc.subcore_barrier()

    _wait_meta(0, 0, msem0)
    _fire_gather(0, rows0, gsem0)
    _fire_gather(1, rows1, gsem1)

    def _chunk(t, rows, rowsf, gsem, ssem, first_pairs):
        blk = lax.div(t, BLK)
        slot = lax.rem(blk, 3)
        ci = lax.rem(t, BLK)
        _drain_gather(rows, gsem)

        @pl.when(jnp.logical_not(first_pairs))
        def _():
            _wait_scatter(rowsf, ssem)
        _scale(slot, ci, rows, rowsf)

        @pl.when(t + 2 < CHUNKS_PER_TILE)
        def _():
            t2 = t + 2
            _fire_gather(t2, rows, gsem)
        _fire_scatter(slot, ci, rowsf, ssem)

    def _pair(p, _):
        t0 = 2 * p
        blk = lax.div(t0, BLK)
        slot = lax.rem(blk, 3)

        @pl.when(lax.rem(t0, BLK) == 0)
        def _():
            _mask(slot)

            @pl.when(blk + 1 < NBLK)
            def _():
                _fire_meta_dyn(blk + 1, lax.rem(blk + 1, 3))

        @pl.when((lax.rem(t0, BLK) == 6) & (blk + 1 < NBLK))
        def _():
            _wait_meta_dyn(blk + 1, lax.rem(blk + 1, 3))

        _chunk(t0, rows0, rowsf0, gsem0, ssem0, p == 0)
        _chunk(t0 + 1, rows1, rowsf1, gsem1, ssem1, p == 0)
        return 0
    lax.fori_loop(0, C2, _pair, 0)

    _wait_scatter(rowsf0, ssem0)
    _wait_scatter(rowsf1, ssem1)
    plsc.subcore_barrier()

    for r in range(ROWS_PER_TILE // WBR):
        row0 = s * ROWS_PER_TILE + r * WBR

        @pl.when((s < NS - 1) | (r < ROWS_PER_TILE // WBR - 1))
        def _():
            pltpu.sync_copy(acc.at[pl.ds(row0, WBR)], wbuf)
            pltpu.sync_copy(wbuf, out_hbm.at[pl.ds(c * HALF + row0, WBR)])

    @pl.when(s == NS - 1)
    def _():
        tail0 = (NS - 1) * ROWS_PER_TILE + (ROWS_PER_TILE // WBR - 1) * WBR
        TAIL = HALF - tail0
        pltpu.sync_copy(acc.at[pl.ds(tail0, TAIL)], wbuf.at[pl.ds(0, TAIL)])
        pltpu.sync_copy(wbuf.at[pl.ds(0, TAIL)],
                        out_hbm.at[pl.ds(c * HALF + tail0, TAIL)])


def _dense_layer(x, h, W1, W2, b):
    RB = 400
    nblk = NN // RB

    def body(x_ref, h_ref, w1_ref, w2_ref, b_ref, o_ref):
        xv = x_ref[...]
        hv = h_ref[...]
        t = ((xv + hv) @ w1_ref[...] + (xv * hv) @ w2_ref[...]
             + b_ref[...])
        t = jnp.where(t >= 0, t, 0.01 * t)
        n = jnp.sqrt(jnp.sum(t * t, axis=1, keepdims=True))
        o_ref[...] = t / jnp.maximum(n, 1e-12)

    return pl.pallas_call(
        body,
        grid=(nblk,),
        in_specs=[
            pl.BlockSpec((RB, D), lambda i: (i, 0)),
            pl.BlockSpec((RB, D), lambda i: (i, 0)),
            pl.BlockSpec((D, D), lambda i: (0, 0)),
            pl.BlockSpec((D, D), lambda i: (0, 0)),
            pl.BlockSpec((1, D), lambda i: (0, 0)),
        ],
        out_specs=pl.BlockSpec((RB, D), lambda i: (i, 0)),
        out_shape=jax.ShapeDtypeStruct((NN, D), jnp.float32),
    )(x, h, W1, W2, b)


_B = 4096
_PT = _B // (NC * NS)


@functools.partial(
    pl.kernel,
    out_type=tuple(
        jax.ShapeDtypeStruct((_B, 4 * D), jnp.uint8) for _ in range(6)),
    mesh=_mesh,
    scratch_types=[
        pltpu.VMEM((_PT,), jnp.int32),
        pltpu.VMEM((_PT,), jnp.int32),
        pltpu.VMEM((_PT, 4 * D), jnp.uint8),
        pltpu.SemaphoreType.DMA,
    ],
    compiler_params=pltpu.CompilerParams(use_tc_tiling_on_sc=False, needs_layout_passes=False),
)
def _gather6(x0, h1, h2, uid, iid, u0o, i0o, u1o, i1o, u2o, i2o,
             uidx, iidx, b64, sem):
    wid = lax.axis_index("s") * NC + lax.axis_index("c")
    base = wid * _PT
    pltpu.sync_copy(uid.at[pl.ds(base, _PT)], uidx)
    pltpu.sync_copy(iid.at[pl.ds(base, _PT)], iidx)
    for tbl, idx, out in ((x0, uidx, u0o), (x0, iidx, i0o),
                          (h1, uidx, u1o), (h1, iidx, i1o),
                          (h2, uidx, u2o), (h2, iidx, i2o)):
        pltpu.async_copy(tbl.at[idx], b64, sem).wait()
        pltpu.sync_copy(b64, out.at[pl.ds(base, _PT)])


def _dot_scores(u0, i0, u1, i1, u2, i2):
    def body(u0r, i0r, u1r, i1r, u2r, i2r, o_ref):
        sc = (jnp.sum(u0r[...] * i0r[...], axis=1)
              + jnp.sum(u1r[...] * i1r[...], axis=1)
              + jnp.sum(u2r[...] * i2r[...], axis=1))
        o_ref[...] = sc[:, None]

    out = pl.pallas_call(
        body,
        out_shape=jax.ShapeDtypeStruct((_B, 1), jnp.float32),
    )(u0, i0, u1, i1, u2, i2)
    return out.reshape(_B)


def _as_u8(x):
    n, d = x.shape
    return lax.bitcast_convert_type(x, jnp.uint8).reshape(n, d * 4)


def _as_f32(x):
    n, d = x.shape
    return lax.bitcast_convert_type(x.reshape(n, d // 4, 4), jnp.float32)


def kernel(user_table, entity_table, W1_0, b1_0, W2_0, b2_0,
           W1_1, b1_1, W2_1, b2_1, edge_vals, edge_index,
           user_ids, item_ids):
    x0 = jnp.concatenate([user_table, entity_table], axis=0)

    src = edge_index[1].astype(jnp.int32)
    dst = edge_index[0].astype(jnp.int32)
    pad = E_PAD - E
    src_p = jnp.concatenate([src, jnp.zeros((pad,), jnp.int32)]).reshape(-1, K)
    dst_p = jnp.concatenate([dst, jnp.full((pad,), NN, jnp.int32)]).reshape(-1, K)
    val_p = jnp.concatenate([edge_vals, jnp.zeros((pad,), jnp.float32)]).reshape(-1, K)

    W1s = jnp.stack([W1_0, jnp.pad(W1_1, ((0, 0), (0, 32)))])
    W2s = jnp.stack([W2_0, jnp.pad(W2_1, ((0, 0), (0, 32)))])
    bs = jnp.stack([(b1_0 + b2_0).reshape(1, -1),
                    jnp.pad((b1_1 + b2_1).reshape(1, -1), ((0, 0), (0, 32)))])

    def _step(h, wb):
        W1, W2, b = wb
        nh = _spmm(src_p, dst_p, val_p, _as_u8(h))
        hn = _dense_layer(h, nh, W1, W2, b)
        return hn, hn

    _, hs = lax.scan(_step, x0, (W1s, W2s, bs))

    uid = user_ids.astype(jnp.int32)
    iid = (item_ids + N_USERS).astype(jnp.int32)
    outs = _gather6(_as_u8(x0), _as_u8(hs[0]), _as_u8(hs[1]), uid, iid)
    u0, i0, u1, i1, u2, i2 = (_as_f32(o) for o in outs)
    return _dot_scores(u0, i0, u1, i1, u2, i2)


# speedup vs baseline: 1.0397x; 1.0397x over previous
import functools

import jax
import jax.numpy as jnp
from jax import lax
from jax.experimental import pallas as pl
from jax.experimental.pallas import tpu as pltpu
from jax.experimental.pallas import tpu_sc as plsc

N_USERS = 10000
N_ENT = 40000
NN = N_USERS + N_ENT
E = 800000
D = 64

NC = 2
NS = 16
HALF = NN // 2
ROWS_PER_TILE = 1564
ACC_ROWS = NS * ROWS_PER_TILE
TRASH = ACC_ROWS - 1

K = 64
CHUNKS_PER_TILE = 784
PER_TILE = CHUNKS_PER_TILE * K
E_PAD = NS * PER_TILE
C2 = CHUNKS_PER_TILE // 2

_mesh = plsc.VectorSubcoreMesh(core_axis_name="c", subcore_axis_name="s")


def _zero16():
    return jnp.zeros((16,), jnp.float32)


BLK = 8
NBLK = CHUNKS_PER_TILE // BLK
WBR = 68


@functools.partial(
    pl.kernel,
    out_type=jax.ShapeDtypeStruct((NN, D), jnp.float32),
    mesh=_mesh,
    scratch_types=[
        pltpu.VMEM((3, BLK, K), jnp.int32),
        pltpu.VMEM((3, BLK, K), jnp.int32),
        pltpu.VMEM((3, BLK, K), jnp.float32),
        pltpu.VMEM((K, 4 * D), jnp.uint8),
        pltpu.VMEM((K, 4 * D), jnp.uint8),
        pltpu.VMEM((K, D), jnp.float32),
        pltpu.VMEM((K, D), jnp.float32),
        pltpu.VMEM((WBR, D), jnp.float32),
        pltpu.VMEM_SHARED((ACC_ROWS, D), jnp.float32),
        pltpu.SemaphoreType.DMA,
        pltpu.SemaphoreType.DMA,
        pltpu.SemaphoreType.DMA,
        pltpu.SemaphoreType.DMA,
        pltpu.SemaphoreType.DMA,
        pltpu.SemaphoreType.DMA,
        pltpu.SemaphoreType.DMA,
    ],
    compiler_params=pltpu.CompilerParams(use_tc_tiling_on_sc=False, needs_layout_passes=False),
)
def _spmm(src_hbm, dst_hbm, val_hbm, feats_hbm, out_hbm,
          srcm, dstm, valm, rows0, rows1, rowsf0, rowsf1, wbuf, acc,
          msem0, msem1, msem2, gsem0, gsem1, ssem0, ssem1):
    c = lax.axis_index("c")
    s = lax.axis_index("s")
    lo = c * HALF
    row_base = s * CHUNKS_PER_TILE
    msems = (msem0, msem1, msem2)

    def _fire_meta(b, slot, sem):
        rb = row_base + b * BLK
        pltpu.async_copy(src_hbm.at[pl.ds(rb, BLK)], srcm.at[slot], sem)
        pltpu.async_copy(dst_hbm.at[pl.ds(rb, BLK)], dstm.at[slot], sem)
        pltpu.async_copy(val_hbm.at[pl.ds(rb, BLK)], valm.at[slot], sem)

    def _wait_meta(b, slot, sem):
        rb = row_base + b * BLK
        pltpu.make_async_copy(src_hbm.at[pl.ds(rb, BLK)], srcm.at[slot], sem).wait()
        pltpu.make_async_copy(dst_hbm.at[pl.ds(rb, BLK)], dstm.at[slot], sem).wait()
        pltpu.make_async_copy(val_hbm.at[pl.ds(rb, BLK)], valm.at[slot], sem).wait()

    def _wait_meta_dyn(b, slot):
        for i in range(3):
            @pl.when(slot == i)
            def _():
                _wait_meta(b, slot, msems[i])

    def _fire_meta_dyn(b, slot):
        for i in range(3):
            @pl.when(slot == i)
            def _():
                _fire_meta(b, slot, msems[i])

    def _mask(slot):
        for ci in range(BLK):
            for v in range(K // 16):
                sl = pl.ds(v * 16, 16)
                dl = dstm[slot, ci, sl] - lo
                inm = (dl >= 0) & (dl < HALF)
                dstm[slot, ci, sl] = jnp.where(inm, dl, TRASH)
                valm[slot, ci, sl] = jnp.where(inm, valm[slot, ci, sl], 0.0)

    def _fire_gather(t, rows, gsem):
        slot = lax.rem(lax.div(t, BLK), 3)
        ci = lax.rem(t, BLK)
        pltpu.async_copy(feats_hbm.at[srcm.at[slot, ci]], rows, gsem)

    def _drain_gather(rows, gsem):
        pltpu.make_async_copy(feats_hbm.at[srcm.at[0, 0]], rows, gsem).wait()

    def _scale(slot, ci, rows, rowsf):
        def _e16(i, _):
            vv = valm[slot, ci, pl.ds(i * 16, 16)]
            for de in range(16):
                e = i * 16 + de
                v = lax.gather(
                    vv, jnp.full((16, 1), de, jnp.int32),
                    lax.GatherDimensionNumbers(
                        offset_dims=(), collapsed_slice_dims=(0,),
                        start_index_map=(0,)),
                    slice_sizes=(1,),
                    mode=lax.GatherScatterMode.PROMISE_IN_BOUNDS)
                for j in range(D // 16):
                    r = plsc.bitcast(rows[e, pl.ds(j * 64, 64)], jnp.float32)
                    rowsf[e, pl.ds(j * 16, 16)] = r * v
            return 0
        lax.fori_loop(0, K // 16, _e16, 0)

    def _fire_scatter(slot, ci, rowsf, ssem):
        pltpu.async_copy(rowsf, acc.at[dstm.at[slot, ci]], ssem, add=True)

    def _wait_scatter(rowsf, ssem):
        pltpu.make_async_copy(rowsf, acc.at[dstm.at[0, 0]], ssem).wait()

    _fire_meta(0, 0, msem0)

    def _fill_z(r, _):
        for j in range(D // 16):
            wbuf[r, pl.ds(j * 16, 16)] = _zero16()
        return 0
    lax.fori_loop(0, WBR, _fill_z, 0)
    for r in range(ROWS_PER_TILE // WBR):
        pltpu.sync_copy(wbuf, acc.at[pl.ds(s * ROWS_PER_TILE + r * WBR, WBR)])
    plsc.subcore_barrier()

    _wait_meta(0, 0, msem0)
    _fire_gather(0, rows0, gsem0)
    _fire_gather(1, rows1, gsem1)

    def _chunk(t, rows, rowsf, gsem, ssem, first_pairs):
        blk = lax.div(t, BLK)
        slot = lax.rem(blk, 3)
        ci = lax.rem(t, BLK)
        _drain_gather(rows, gsem)

        @pl.when(jnp.logical_not(first_pairs))
        def _():
            _wait_scatter(rowsf, ssem)
        _scale(slot, ci, rows, rowsf)

        @pl.when(t + 2 < CHUNKS_PER_TILE)
        def _():
            t2 = t + 2
            _fire_gather(t2, rows, gsem)
        _fire_scatter(slot, ci, rowsf, ssem)

    def _pair(p, _):
        t0 = 2 * p
        blk = lax.div(t0, BLK)
        slot = lax.rem(blk, 3)

        @pl.when(lax.rem(t0, BLK) == 0)
        def _():
            _mask(slot)

            @pl.when(blk + 1 < NBLK)
            def _():
                _fire_meta_dyn(blk + 1, lax.rem(blk + 1, 3))

        @pl.when((lax.rem(t0, BLK) == 6) & (blk + 1 < NBLK))
        def _():
            _wait_meta_dyn(blk + 1, lax.rem(blk + 1, 3))

        _chunk(t0, rows0, rowsf0, gsem0, ssem0, p == 0)
        _chunk(t0 + 1, rows1, rowsf1, gsem1, ssem1, p == 0)
        return 0
    lax.fori_loop(0, C2, _pair, 0)

    _wait_scatter(rowsf0, ssem0)
    _wait_scatter(rowsf1, ssem1)
    plsc.subcore_barrier()

    for r in range(ROWS_PER_TILE // WBR):
        row0 = s * ROWS_PER_TILE + r * WBR

        @pl.when((s < NS - 1) | (r < ROWS_PER_TILE // WBR - 1))
        def _():
            pltpu.sync_copy(acc.at[pl.ds(row0, WBR)], wbuf)
            pltpu.sync_copy(wbuf, out_hbm.at[pl.ds(c * HALF + row0, WBR)])

    @pl.when(s == NS - 1)
    def _():
        tail0 = (NS - 1) * ROWS_PER_TILE + (ROWS_PER_TILE // WBR - 1) * WBR
        TAIL = HALF - tail0
        pltpu.sync_copy(acc.at[pl.ds(tail0, TAIL)], wbuf.at[pl.ds(0, TAIL)])
        pltpu.sync_copy(wbuf.at[pl.ds(0, TAIL)],
                        out_hbm.at[pl.ds(c * HALF + tail0, TAIL)])


def _dense_layer(x, h, W1, W2, b):
    RB = 2000
    nblk = NN // RB

    def body(x_ref, h_ref, w1_ref, w2_ref, b_ref, o_ref):
        xv = x_ref[...]
        hv = h_ref[...]
        t = ((xv + hv) @ w1_ref[...] + (xv * hv) @ w2_ref[...]
             + b_ref[...])
        t = jnp.where(t >= 0, t, 0.01 * t)
        n = jnp.sqrt(jnp.sum(t * t, axis=1, keepdims=True))
        o_ref[...] = t / jnp.maximum(n, 1e-12)

    return pl.pallas_call(
        body,
        grid=(nblk,),
        in_specs=[
            pl.BlockSpec((RB, D), lambda i: (i, 0)),
            pl.BlockSpec((RB, D), lambda i: (i, 0)),
            pl.BlockSpec((D, D), lambda i: (0, 0)),
            pl.BlockSpec((D, D), lambda i: (0, 0)),
            pl.BlockSpec((1, D), lambda i: (0, 0)),
        ],
        out_specs=pl.BlockSpec((RB, D), lambda i: (i, 0)),
        out_shape=jax.ShapeDtypeStruct((NN, D), jnp.float32),
    )(x, h, W1, W2, b)


_B = 4096
_PT = _B // (NC * NS)


@functools.partial(
    pl.kernel,
    out_type=tuple(
        jax.ShapeDtypeStruct((_B, 4 * D), jnp.uint8) for _ in range(6)),
    mesh=_mesh,
    scratch_types=[
        pltpu.VMEM((_PT,), jnp.int32),
        pltpu.VMEM((_PT,), jnp.int32),
        pltpu.VMEM((_PT, 4 * D), jnp.uint8),
        pltpu.VMEM((_PT, 4 * D), jnp.uint8),
        pltpu.VMEM((_PT, 4 * D), jnp.uint8),
        pltpu.VMEM((_PT, 4 * D), jnp.uint8),
        pltpu.VMEM((_PT, 4 * D), jnp.uint8),
        pltpu.VMEM((_PT, 4 * D), jnp.uint8),
        pltpu.SemaphoreType.DMA,
        pltpu.SemaphoreType.DMA,
        pltpu.SemaphoreType.DMA,
        pltpu.SemaphoreType.DMA,
        pltpu.SemaphoreType.DMA,
        pltpu.SemaphoreType.DMA,
    ],
    compiler_params=pltpu.CompilerParams(use_tc_tiling_on_sc=False, needs_layout_passes=False),
)
def _gather6(x0, h1, h2, uid, iid, u0o, i0o, u1o, i1o, u2o, i2o,
             uidx, iidx, b0, b1, b2, b3, b4, b5, s0, s1, s2, s3, s4, s5):
    wid = lax.axis_index("s") * NC + lax.axis_index("c")
    base = wid * _PT
    pltpu.sync_copy(uid.at[pl.ds(base, _PT)], uidx)
    pltpu.sync_copy(iid.at[pl.ds(base, _PT)], iidx)
    plan = ((x0, uidx, u0o, b0, s0), (x0, iidx, i0o, b1, s1),
            (h1, uidx, u1o, b2, s2), (h1, iidx, i1o, b3, s3),
            (h2, uidx, u2o, b4, s4), (h2, iidx, i2o, b5, s5))
    for tbl, idx, out, buf, sem in plan:
        pltpu.async_copy(tbl.at[idx], buf, sem)
    for tbl, idx, out, buf, sem in plan:
        pltpu.make_async_copy(tbl.at[idx], buf, sem).wait()
        pltpu.sync_copy(buf, out.at[pl.ds(base, _PT)])


def _dot_scores(u0, i0, u1, i1, u2, i2):
    def body(u0r, i0r, u1r, i1r, u2r, i2r, o_ref):
        sc = (jnp.sum(u0r[...] * i0r[...], axis=1)
              + jnp.sum(u1r[...] * i1r[...], axis=1)
              + jnp.sum(u2r[...] * i2r[...], axis=1))
        o_ref[...] = sc[:, None]

    out = pl.pallas_call(
        body,
        out_shape=jax.ShapeDtypeStruct((_B, 1), jnp.float32),
    )(u0, i0, u1, i1, u2, i2)
    return out.reshape(_B)


def _as_u8(x):
    n, d = x.shape
    return lax.bitcast_convert_type(x, jnp.uint8).reshape(n, d * 4)


def _as_f32(x):
    n, d = x.shape
    return lax.bitcast_convert_type(x.reshape(n, d // 4, 4), jnp.float32)


def kernel(user_table, entity_table, W1_0, b1_0, W2_0, b2_0,
           W1_1, b1_1, W2_1, b2_1, edge_vals, edge_index,
           user_ids, item_ids):
    x0 = jnp.concatenate([user_table, entity_table], axis=0)

    src = edge_index[1].astype(jnp.int32)
    dst = edge_index[0].astype(jnp.int32)
    pad = E_PAD - E
    src_p = jnp.concatenate([src, jnp.zeros((pad,), jnp.int32)]).reshape(-1, K)
    dst_p = jnp.concatenate([dst, jnp.full((pad,), NN, jnp.int32)]).reshape(-1, K)
    val_p = jnp.concatenate([edge_vals, jnp.zeros((pad,), jnp.float32)]).reshape(-1, K)

    W1s = jnp.stack([W1_0, jnp.pad(W1_1, ((0, 0), (0, 32)))])
    W2s = jnp.stack([W2_0, jnp.pad(W2_1, ((0, 0), (0, 32)))])
    bs = jnp.stack([(b1_0 + b2_0).reshape(1, -1),
                    jnp.pad((b1_1 + b2_1).reshape(1, -1), ((0, 0), (0, 32)))])

    def _step(h, wb):
        W1, W2, b = wb
        nh = _spmm(src_p, dst_p, val_p, _as_u8(h))
        hn = _dense_layer(h, nh, W1, W2, b)
        return hn, hn

    _, hs = lax.scan(_step, x0, (W1s, W2s, bs))

    uid = user_ids.astype(jnp.int32)
    iid = (item_ids + N_USERS).astype(jnp.int32)
    outs = _gather6(_as_u8(x0), _as_u8(hs[0]), _as_u8(hs[1]), uid, iid)
    u0, i0, u1, i1, u2, i2 = (_as_f32(o) for o in outs)
    return _dot_scores(u0, i0, u1, i1, u2, i2)
